# Initial kernel scaffold; baseline (speedup 1.0000x reference)
import jax
import jax.numpy as jnp
from jax.experimental import pallas as pl


def kernel(states_action, states_graph_ids, states_first, states_second, sates_num_edges, W_msg, b_msg, W_s1, b_s1, W_s2, b_s2, W_r1, b_r1, W_r2, b_r2, W_r3, b_r3):
    raise NotImplementedError("write your pallas kernel here")


# speedup vs baseline: 3.7334x; 3.7334x over previous
import functools

import jax
import jax.numpy as jnp
from jax import lax
from jax.experimental import pallas as pl
from jax.experimental.pallas import tpu as pltpu
from jax.experimental.pallas import tpu_sc as plsc

N = 10000
E = 320000
D = 128
G = 64
R = 256
T = 4

BN = 1024
N_PAD = 10240
DUMMY = N

NC = 2
NS = 16
NW = NC * NS
K = 128
NB = -(-E // (NW * K))
E_PAD = NW * NB * K

_SELU_SCALE = 1.0507009873554805
_SELU_ALPHA = 1.6732632423543772
_SA = _SELU_SCALE * _SELU_ALPHA


def _selu(x):
    return (_SELU_SCALE * jnp.maximum(x, 0.0)
            + (_SA * jnp.exp(jnp.minimum(x, 0.0)) - _SA))


def _edge_pass_body(u_hbm, v_hbm, first_hbm, second_hbm, zeros_hbm, out_hbm,
                    idx1, idx2, urows, vrows, acc, sem1, sem2):
    cid = lax.axis_index("c")
    sid = lax.axis_index("s")
    wid = sid * NC + cid

    rpt = N_PAD // NS
    pltpu.sync_copy(zeros_hbm.at[pl.ds(sid * rpt, rpt)],
                    acc.at[pl.ds(sid * rpt, rpt)])
    plsc.subcore_barrier()

    def batch_body(b, _):
        base = (wid * NB + b) * K
        pltpu.sync_copy(first_hbm.at[pl.ds(base, K)], idx1)
        pltpu.sync_copy(second_hbm.at[pl.ds(base, K)], idx2)
        cp1 = pltpu.async_copy(u_hbm.at[idx1], urows, sem1)
        cp2 = pltpu.async_copy(v_hbm.at[idx2], vrows, sem2)
        cp1.wait()
        cp2.wait()

        def row_body(r, _):
            for c in range(0, D, 16):
                x = urows[r, pl.ds(c, 16)] + vrows[r, pl.ds(c, 16)]
                e = jnp.exp(jnp.minimum(x, 0.0))
                urows[r, pl.ds(c, 16)] = (
                    _SELU_SCALE * jnp.maximum(x, 0.0) + (_SA * e - _SA))
            return 0

        lax.fori_loop(0, K, row_body, 0, unroll=False)
        pltpu.sync_copy(urows, acc.at[idx2], add=True)
        return 0

    lax.fori_loop(0, NB, batch_body, 0, unroll=False)
    plsc.subcore_barrier()
    pltpu.sync_copy(acc.at[pl.ds(sid * rpt, rpt)],
                    out_hbm.at[pl.ds(cid * N_PAD + sid * rpt, rpt)])


_edge_pass_cached = None


def _edge_pass(*args):
    global _edge_pass_cached
    if _edge_pass_cached is None:
        mesh = plsc.VectorSubcoreMesh(core_axis_name="c",
                                      subcore_axis_name="s")
        _edge_pass_cached = pl.kernel(
            _edge_pass_body,
            out_type=jax.ShapeDtypeStruct((NC * N_PAD, D), jnp.float32),
            mesh=mesh,
            scratch_types=[
                pltpu.VMEM((K,), jnp.int32),
                pltpu.VMEM((K,), jnp.int32),
                pltpu.VMEM((K, D), jnp.float32),
                pltpu.VMEM((K, D), jnp.float32),
                pltpu.VMEM_SHARED((N_PAD, D), jnp.float32),
                pltpu.SemaphoreType.DMA,
                pltpu.SemaphoreType.DMA,
            ],
        )
    return _edge_pass_cached(*args)


def _uv_body(ls_ref, wcat_ref, bmsg_ref, u_ref, v_ref):
    uv = jnp.dot(ls_ref[...], wcat_ref[...],
                 preferred_element_type=jnp.float32)
    u_ref[...] = uv[:, :D] + bmsg_ref[...]
    v_ref[...] = uv[:, D:]


def _uv_call(ls, wcat, bmsg):
    return pl.pallas_call(
        _uv_body,
        grid=(N_PAD // BN,),
        in_specs=[
            pl.BlockSpec((BN, D), lambda i: (i, 0)),
            pl.BlockSpec((D, 2 * D), lambda i: (0, 0)),
            pl.BlockSpec((1, D), lambda i: (0, 0)),
        ],
        out_specs=[
            pl.BlockSpec((BN, D), lambda i: (i, 0)),
            pl.BlockSpec((BN, D), lambda i: (i, 0)),
        ],
        out_shape=[
            jax.ShapeDtypeStruct((N_PAD, D), jnp.float32),
            jax.ShapeDtypeStruct((N_PAD, D), jnp.float32),
        ],
    )(ls, wcat, bmsg)


def _node_body(ls_ref, agga_ref, aggb_ref, w1t_ref, w1b_ref, b1_ref,
               w2_ref, b2_ref, wcat_ref, bmsg_ref,
               ls_out, u_out, v_out):
    agg = agga_ref[...] + aggb_ref[...]
    h = _selu(jnp.dot(ls_ref[...], w1t_ref[...],
                      preferred_element_type=jnp.float32)
              + jnp.dot(agg, w1b_ref[...],
                        preferred_element_type=jnp.float32)
              + b1_ref[...])
    ls_new = _selu(jnp.dot(h, w2_ref[...],
                           preferred_element_type=jnp.float32) + b2_ref[...])
    ls_out[...] = ls_new
    uv = jnp.dot(ls_new, wcat_ref[...], preferred_element_type=jnp.float32)
    u_out[...] = uv[:, :D] + bmsg_ref[...]
    v_out[...] = uv[:, D:]


def _node_call(ls, agg2, w1t, w1b, b1, w2, b2, wcat, bmsg):
    nb = N_PAD // BN
    return pl.pallas_call(
        _node_body,
        grid=(nb,),
        in_specs=[
            pl.BlockSpec((BN, D), lambda i: (i, 0)),
            pl.BlockSpec((BN, D), lambda i: (i, 0)),
            pl.BlockSpec((BN, D), lambda i, _nb=nb: (_nb + i, 0)),
            pl.BlockSpec((D, D), lambda i: (0, 0)),
            pl.BlockSpec((D, D), lambda i: (0, 0)),
            pl.BlockSpec((1, D), lambda i: (0, 0)),
            pl.BlockSpec((D, D), lambda i: (0, 0)),
            pl.BlockSpec((1, D), lambda i: (0, 0)),
            pl.BlockSpec((D, 2 * D), lambda i: (0, 0)),
            pl.BlockSpec((1, D), lambda i: (0, 0)),
        ],
        out_specs=[
            pl.BlockSpec((BN, D), lambda i: (i, 0)),
            pl.BlockSpec((BN, D), lambda i: (i, 0)),
            pl.BlockSpec((BN, D), lambda i: (i, 0)),
        ],
        out_shape=[
            jax.ShapeDtypeStruct((N_PAD, D), jnp.float32),
            jax.ShapeDtypeStruct((N_PAD, D), jnp.float32),
            jax.ShapeDtypeStruct((N_PAD, D), jnp.float32),
        ],
    )(ls, agg2, agg2, w1t, w1b, b1, w2, b2, wcat, bmsg)


def _readout_body(ls_ref, gid_ref, wr1_ref, br1_ref, wr2_ref, br2_ref,
                  wr3_ref, out_ref, acc_ref):
    i = pl.program_id(0)

    @pl.when(i == 0)
    def _init():
        acc_ref[...] = jnp.zeros_like(acc_ref)

    ids = gid_ref[0]
    onehot = (lax.broadcasted_iota(jnp.int32, (G, BN), 0) == ids
              ).astype(jnp.float32)
    acc_ref[...] += jnp.dot(onehot, ls_ref[...],
                            preferred_element_type=jnp.float32)

    @pl.when(i == pl.num_programs(0) - 1)
    def _fin():
        r = _selu(jnp.dot(acc_ref[...], wr1_ref[...],
                          preferred_element_type=jnp.float32) + br1_ref[...])
        r = _selu(jnp.dot(r, wr2_ref[...],
                          preferred_element_type=jnp.float32) + br2_ref[...])
        out_ref[...] = jnp.sum(r * wr3_ref[...], axis=1, keepdims=True) + \
            jnp.zeros((G, D), jnp.float32)


def _readout_call(ls, gid3, wr1, br1, wr2, br2, wr3row):
    return pl.pallas_call(
        _readout_body,
        grid=(N_PAD // BN,),
        in_specs=[
            pl.BlockSpec((BN, D), lambda i: (i, 0)),
            pl.BlockSpec((1, 1, BN), lambda i: (i, 0, 0)),
            pl.BlockSpec((D, R), lambda i: (0, 0)),
            pl.BlockSpec((1, R), lambda i: (0, 0)),
            pl.BlockSpec((R, R), lambda i: (0, 0)),
            pl.BlockSpec((1, R), lambda i: (0, 0)),
            pl.BlockSpec((1, R), lambda i: (0, 0)),
        ],
        out_specs=pl.BlockSpec((G, D), lambda i: (0, 0)),
        out_shape=jax.ShapeDtypeStruct((G, D), jnp.float32),
        scratch_shapes=[pltpu.VMEM((G, D), jnp.float32)],
    )(ls, gid3, wr1, br1, wr2, br2, wr3row)


def kernel(states_action, states_graph_ids, states_first, states_second,
           sates_num_edges, W_msg, b_msg, W_s1, b_s1, W_s2, b_s2,
           W_r1, b_r1, W_r2, b_r2, W_r3, b_r3):
    ls = jnp.pad(states_action, ((0, N_PAD - N), (0, 0)))
    first_p = jnp.concatenate(
        [states_first, jnp.zeros((E_PAD - E,), jnp.int32)])
    second_p = jnp.concatenate(
        [states_second, jnp.full((E_PAD - E,), DUMMY, jnp.int32)])
    gid3 = jnp.pad(states_graph_ids, (0, N_PAD - N),
                   constant_values=G).reshape(N_PAD // BN, 1, BN)
    zeros = jnp.zeros((N_PAD, D), jnp.float32)

    wcat = jnp.concatenate([W_msg[:D], W_msg[D:]], axis=1)
    bmsg = b_msg.reshape(1, D)
    w1t, w1b = W_s1[:D], W_s1[D:]
    b1 = b_s1.reshape(1, D)
    b2 = b_s2.reshape(1, D)
    br1 = b_r1.reshape(1, R)
    br2 = b_r2.reshape(1, R)
    wr3row = W_r3.reshape(1, R)

    u, v = _uv_call(ls, wcat, bmsg)
    for _ in range(T):
        agg2 = _edge_pass(u, v, first_p, second_p, zeros)
        ls, u, v = _node_call(ls, agg2, w1t, w1b, b1, W_s2, b2, wcat, bmsg)

    out = _readout_call(ls, gid3, W_r1, br1, W_r2, br2, wr3row)
    r = out[:, :1] + b_r3
    return r + 0.0 * jnp.asarray(sates_num_edges, dtype=r.dtype)


# speedup vs baseline: 5.2495x; 1.4061x over previous
import functools

import jax
import jax.numpy as jnp
from jax import lax
from jax.experimental import pallas as pl
from jax.experimental.pallas import tpu as pltpu
from jax.experimental.pallas import tpu_sc as plsc

N = 10000
E = 320000
D = 128
G = 64
R = 256
T = 4

BN = 1024
N_PAD = 10240
DUMMY = N

NC = 2
NS = 16
NW = NC * NS
K = 64
NB = 157
E_PAD = NW * NB * K
assert NB * K * NW >= E and NB % 2 == 1

_SELU_SCALE = 1.0507009873554805
_SELU_ALPHA = 1.6732632423543772
_SA = _SELU_SCALE * _SELU_ALPHA


def _selu(x):
    return (_SELU_SCALE * jnp.maximum(x, 0.0)
            + (_SA * jnp.exp(jnp.minimum(x, 0.0)) - _SA))


def _edge_pass_body(u_hbm, v_hbm, first_hbm, second_hbm, zeros_hbm, out_hbm,
                    i1b0, i1b1, i2b0, i2b1, ub0, ub1, vb0, vb1,
                    acc, gsem0, gsem1):
    cid = lax.axis_index("c")
    sid = lax.axis_index("s")
    wid = sid * NC + cid
    ubufs = (ub0, ub1)
    vbufs = (vb0, vb1)
    i1bufs = (i1b0, i1b1)
    i2bufs = (i2b0, i2b1)
    gsems = (gsem0, gsem1)

    rpt = N_PAD // NS
    pltpu.sync_copy(zeros_hbm.at[pl.ds(sid * rpt, rpt)],
                    acc.at[pl.ds(sid * rpt, rpt)])
    ebase = wid * (NB * K)
    plsc.subcore_barrier()

    def issue_gather(b, par):
        base = ebase + b * K
        pltpu.sync_copy(first_hbm.at[pl.ds(base, K)], i1bufs[par])
        pltpu.sync_copy(second_hbm.at[pl.ds(base, K)], i2bufs[par])
        pltpu.async_copy(u_hbm.at[i1bufs[par]], ubufs[par], gsems[par])
        pltpu.async_copy(v_hbm.at[i2bufs[par]], vbufs[par], gsems[par])

    def wait_gathers(par):
        pltpu.make_async_copy(u_hbm.at[i2bufs[par]], ubufs[par],
                              gsems[par]).wait()
        pltpu.make_async_copy(v_hbm.at[i2bufs[par]], vbufs[par],
                              gsems[par]).wait()

    def do_batch(par):
        urows, vrows = ubufs[par], vbufs[par]

        def row_body(r, _):
            for c in range(0, D, 16):
                x = urows[r, pl.ds(c, 16)] + vrows[r, pl.ds(c, 16)]
                e = jnp.exp(jnp.minimum(x, 0.0))
                urows[r, pl.ds(c, 16)] = (
                    _SELU_SCALE * jnp.maximum(x, 0.0) + (_SA * e - _SA))
            return 0

        lax.fori_loop(0, K, row_body, 0, unroll=False)
        pltpu.sync_copy(urows, acc.at[i2bufs[par]], add=True)

    issue_gather(0, 0)

    def pair_body(i, _):
        g0 = i * 2
        wait_gathers(0)
        issue_gather(g0 + 1, 1)
        do_batch(0)
        wait_gathers(1)
        issue_gather(g0 + 2, 0)
        do_batch(1)
        return 0

    lax.fori_loop(0, (NB - 1) // 2, pair_body, 0, unroll=False)
    wait_gathers(0)
    do_batch(0)

    plsc.subcore_barrier()
    pltpu.sync_copy(acc.at[pl.ds(sid * rpt, rpt)],
                    out_hbm.at[pl.ds(cid * N_PAD + sid * rpt, rpt)])


_edge_pass_cached = None


def _edge_pass(*args):
    global _edge_pass_cached
    if _edge_pass_cached is None:
        mesh = plsc.VectorSubcoreMesh(core_axis_name="c",
                                      subcore_axis_name="s")
        _edge_pass_cached = pl.kernel(
            _edge_pass_body,
            out_type=jax.ShapeDtypeStruct((NC * N_PAD, D), jnp.float32),
            mesh=mesh,
            scratch_types=[
                pltpu.VMEM((K,), jnp.int32),
                pltpu.VMEM((K,), jnp.int32),
                pltpu.VMEM((K,), jnp.int32),
                pltpu.VMEM((K,), jnp.int32),
                pltpu.VMEM((K, D), jnp.float32),
                pltpu.VMEM((K, D), jnp.float32),
                pltpu.VMEM((K, D), jnp.float32),
                pltpu.VMEM((K, D), jnp.float32),
                pltpu.VMEM_SHARED((N_PAD, D), jnp.float32),
                pltpu.SemaphoreType.DMA,
                pltpu.SemaphoreType.DMA,
            ],
        )
    return _edge_pass_cached(*args)


def _uv_body(ls_ref, wcat_ref, bmsg_ref, u_ref, v_ref):
    uv = jnp.dot(ls_ref[...], wcat_ref[...],
                 preferred_element_type=jnp.float32)
    u_ref[...] = uv[:, :D] + bmsg_ref[...]
    v_ref[...] = uv[:, D:]


def _uv_call(ls, wcat, bmsg):
    return pl.pallas_call(
        _uv_body,
        grid=(N_PAD // BN,),
        in_specs=[
            pl.BlockSpec((BN, D), lambda i: (i, 0)),
            pl.BlockSpec((D, 2 * D), lambda i: (0, 0)),
            pl.BlockSpec((1, D), lambda i: (0, 0)),
        ],
        out_specs=[
            pl.BlockSpec((BN, D), lambda i: (i, 0)),
            pl.BlockSpec((BN, D), lambda i: (i, 0)),
        ],
        out_shape=[
            jax.ShapeDtypeStruct((N_PAD, D), jnp.float32),
            jax.ShapeDtypeStruct((N_PAD, D), jnp.float32),
        ],
    )(ls, wcat, bmsg)


def _node_body(ls_ref, agga_ref, aggb_ref, w1t_ref, w1b_ref, b1_ref,
               w2_ref, b2_ref, wcat_ref, bmsg_ref,
               ls_out, u_out, v_out):
    agg = agga_ref[...] + aggb_ref[...]
    h = _selu(jnp.dot(ls_ref[...], w1t_ref[...],
                      preferred_element_type=jnp.float32)
              + jnp.dot(agg, w1b_ref[...],
                        preferred_element_type=jnp.float32)
              + b1_ref[...])
    ls_new = _selu(jnp.dot(h, w2_ref[...],
                           preferred_element_type=jnp.float32) + b2_ref[...])
    ls_out[...] = ls_new
    uv = jnp.dot(ls_new, wcat_ref[...], preferred_element_type=jnp.float32)
    u_out[...] = uv[:, :D] + bmsg_ref[...]
    v_out[...] = uv[:, D:]


def _node_call(ls, agg2, w1t, w1b, b1, w2, b2, wcat, bmsg):
    nb = N_PAD // BN
    return pl.pallas_call(
        _node_body,
        grid=(nb,),
        in_specs=[
            pl.BlockSpec((BN, D), lambda i: (i, 0)),
            pl.BlockSpec((BN, D), lambda i: (i, 0)),
            pl.BlockSpec((BN, D), lambda i, _nb=nb: (_nb + i, 0)),
            pl.BlockSpec((D, D), lambda i: (0, 0)),
            pl.BlockSpec((D, D), lambda i: (0, 0)),
            pl.BlockSpec((1, D), lambda i: (0, 0)),
            pl.BlockSpec((D, D), lambda i: (0, 0)),
            pl.BlockSpec((1, D), lambda i: (0, 0)),
            pl.BlockSpec((D, 2 * D), lambda i: (0, 0)),
            pl.BlockSpec((1, D), lambda i: (0, 0)),
        ],
        out_specs=[
            pl.BlockSpec((BN, D), lambda i: (i, 0)),
            pl.BlockSpec((BN, D), lambda i: (i, 0)),
            pl.BlockSpec((BN, D), lambda i: (i, 0)),
        ],
        out_shape=[
            jax.ShapeDtypeStruct((N_PAD, D), jnp.float32),
            jax.ShapeDtypeStruct((N_PAD, D), jnp.float32),
            jax.ShapeDtypeStruct((N_PAD, D), jnp.float32),
        ],
    )(ls, agg2, agg2, w1t, w1b, b1, w2, b2, wcat, bmsg)


def _readout_body(ls_ref, gid_ref, wr1_ref, br1_ref, wr2_ref, br2_ref,
                  wr3_ref, out_ref, acc_ref):
    i = pl.program_id(0)

    @pl.when(i == 0)
    def _init():
        acc_ref[...] = jnp.zeros_like(acc_ref)

    ids = gid_ref[0]
    onehot = (lax.broadcasted_iota(jnp.int32, (G, BN), 0) == ids
              ).astype(jnp.float32)
    acc_ref[...] += jnp.dot(onehot, ls_ref[...],
                            preferred_element_type=jnp.float32)

    @pl.when(i == pl.num_programs(0) - 1)
    def _fin():
        r = _selu(jnp.dot(acc_ref[...], wr1_ref[...],
                          preferred_element_type=jnp.float32) + br1_ref[...])
        r = _selu(jnp.dot(r, wr2_ref[...],
                          preferred_element_type=jnp.float32) + br2_ref[...])
        out_ref[...] = jnp.sum(r * wr3_ref[...], axis=1, keepdims=True) + \
            jnp.zeros((G, D), jnp.float32)


def _readout_call(ls, gid3, wr1, br1, wr2, br2, wr3row):
    return pl.pallas_call(
        _readout_body,
        grid=(N_PAD // BN,),
        in_specs=[
            pl.BlockSpec((BN, D), lambda i: (i, 0)),
            pl.BlockSpec((1, 1, BN), lambda i: (i, 0, 0)),
            pl.BlockSpec((D, R), lambda i: (0, 0)),
            pl.BlockSpec((1, R), lambda i: (0, 0)),
            pl.BlockSpec((R, R), lambda i: (0, 0)),
            pl.BlockSpec((1, R), lambda i: (0, 0)),
            pl.BlockSpec((1, R), lambda i: (0, 0)),
        ],
        out_specs=pl.BlockSpec((G, D), lambda i: (0, 0)),
        out_shape=jax.ShapeDtypeStruct((G, D), jnp.float32),
        scratch_shapes=[pltpu.VMEM((G, D), jnp.float32)],
    )(ls, gid3, wr1, br1, wr2, br2, wr3row)


def kernel(states_action, states_graph_ids, states_first, states_second,
           sates_num_edges, W_msg, b_msg, W_s1, b_s1, W_s2, b_s2,
           W_r1, b_r1, W_r2, b_r2, W_r3, b_r3):
    ls = jnp.pad(states_action, ((0, N_PAD - N), (0, 0)))
    first_p = jnp.concatenate(
        [states_first, jnp.zeros((E_PAD - E,), jnp.int32)])
    second_p = jnp.concatenate(
        [states_second, jnp.full((E_PAD - E,), DUMMY, jnp.int32)])
    gid3 = jnp.pad(states_graph_ids, (0, N_PAD - N),
                   constant_values=G).reshape(N_PAD // BN, 1, BN)
    zeros = jnp.zeros((N_PAD, D), jnp.float32)

    wcat = jnp.concatenate([W_msg[:D], W_msg[D:]], axis=1)
    bmsg = b_msg.reshape(1, D)
    w1t, w1b = W_s1[:D], W_s1[D:]
    b1 = b_s1.reshape(1, D)
    b2 = b_s2.reshape(1, D)
    br1 = b_r1.reshape(1, R)
    br2 = b_r2.reshape(1, R)
    wr3row = W_r3.reshape(1, R)

    u, v = _uv_call(ls, wcat, bmsg)
    for _ in range(T):
        agg2 = _edge_pass(u, v, first_p, second_p, zeros)
        ls, u, v = _node_call(ls, agg2, w1t, w1b, b1, W_s2, b2, wcat, bmsg)

    out = _readout_call(ls, gid3, W_r1, br1, W_r2, br2, wr3row)
    r = out[:, :1] + b_r3
    return r + 0.0 * jnp.asarray(sates_num_edges, dtype=r.dtype)


# speedup vs baseline: 7.5044x; 1.4296x over previous
import functools

import jax
import jax.numpy as jnp
from jax import lax
from jax.experimental import pallas as pl
from jax.experimental.pallas import tpu as pltpu
from jax.experimental.pallas import tpu_sc as plsc

N = 10000
E = 320000
D = 128
G = 64
R = 256
T = 4

BN = 1024
N_PAD = 10240
DUMMY = N

NC = 2
NS = 16
NW = NC * NS
K = 64
NB = 157
E_PAD = NW * NB * K
N_ACC = 10112
assert NB * K * NW >= E and (NB - 4) % 3 == 0

_SELU_SCALE = 1.0507009873554805
_SELU_ALPHA = 1.6732632423543772
_SA = _SELU_SCALE * _SELU_ALPHA


def _selu(x):
    return (_SELU_SCALE * jnp.maximum(x, 0.0)
            + (_SA * jnp.exp(jnp.minimum(x, 0.0)) - _SA))


def _edge_pass_body(u_hbm, v_hbm, eidx_hbm, zeros_hbm, out_hbm,
                    ib0, ib1, ib2, sb0, sb1, sb2,
                    ub0, ub1, ub2, vb0, vb1, vb2,
                    acc, isem0, isem1, isem2, gsem0, gsem1, gsem2,
                    ssem0, ssem1, ssem2):
    cid = lax.axis_index("c")
    sid = lax.axis_index("s")
    wid = sid * NC + cid
    ib = (ib0, ib1, ib2)
    sb = (sb0, sb1, sb2)
    ub = (ub0, ub1, ub2)
    vb = (vb0, vb1, vb2)
    isem = (isem0, isem1, isem2)
    gsem = (gsem0, gsem1, gsem2)
    ssem = (ssem0, ssem1, ssem2)

    rpt = N_ACC // NS
    pltpu.sync_copy(zeros_hbm.at[pl.ds(sid * rpt, rpt)],
                    acc.at[pl.ds(sid * rpt, rpt)])
    wb = wid * NB

    def fetch_idx(b, s):
        b = jnp.minimum(b, NB - 1)
        pltpu.async_copy(eidx_hbm.at[pl.ds((wb + b) * (2 * K), 2 * K)],
                         ib[s], isem[s])

    def wait_idx(s):
        pltpu.make_async_copy(eidx_hbm.at[pl.ds(0, 2 * K)], ib[s],
                              isem[s]).wait()

    def issue_gathers(s):
        for c in range(0, K, 16):
            sb[s][pl.ds(c, 16)] = ib[s][pl.ds(K + c, 16)]
        pltpu.async_copy(u_hbm.at[ib[s].at[pl.ds(0, K)]], ub[s], gsem[s])
        pltpu.async_copy(v_hbm.at[sb[s]], vb[s], gsem[s])

    def wait_gathers(s):
        pltpu.make_async_copy(u_hbm.at[sb[s]], ub[s], gsem[s]).wait()
        pltpu.make_async_copy(v_hbm.at[sb[s]], vb[s], gsem[s]).wait()

    def compute(s):
        urows, vrows = ub[s], vb[s]

        def row_body(r, _):
            for c in range(0, D, 16):
                x = urows[r, pl.ds(c, 16)] + vrows[r, pl.ds(c, 16)]
                e = jnp.exp(jnp.minimum(x, 0.0))
                urows[r, pl.ds(c, 16)] = (
                    _SELU_SCALE * jnp.maximum(x, 0.0) + (_SA * e - _SA))
            return 0

        lax.fori_loop(0, K, row_body, 0, unroll=False)

    def scatter(s):
        pltpu.async_copy(ub[s], acc.at[sb[s]], ssem[s], add=True)

    def wait_scatter(s):
        pltpu.make_async_copy(ub[s], acc.at[sb[s]], ssem[s]).wait()

    fetch_idx(jnp.int32(0), 0)
    fetch_idx(jnp.int32(1), 1)
    wait_idx(0)
    issue_gathers(0)
    fetch_idx(jnp.int32(2), 2)
    wait_idx(1)
    issue_gathers(1)
    wait_gathers(0)
    fetch_idx(jnp.int32(3), 0)
    wait_idx(2)
    issue_gathers(2)
    compute(0)
    scatter(0)
    wait_gathers(1)
    fetch_idx(jnp.int32(4), 1)
    compute(1)
    scatter(1)

    def tri_body(i, _):
        b0 = 2 + 3 * i
        for j in range(3):
            b = b0 + j
            s = (2 + j) % 3
            sn = (s + 1) % 3
            wait_gathers(s)
            fetch_idx(b + 3, s)
            wait_idx(sn)
            wait_scatter(sn)
            issue_gathers(sn)
            compute(s)
            scatter(s)
        return 0

    lax.fori_loop(0, (NB - 4) // 3, tri_body, 0, unroll=False)

    wait_gathers(2)
    wait_idx(0)
    wait_scatter(0)
    issue_gathers(0)
    compute(2)
    scatter(2)
    wait_gathers(0)
    compute(0)
    scatter(0)
    wait_scatter(0)
    wait_scatter(1)
    wait_scatter(2)
    wait_idx(1)

    plsc.subcore_barrier()
    pltpu.sync_copy(acc.at[pl.ds(sid * rpt, rpt)],
                    out_hbm.at[pl.ds(cid * N_PAD + sid * rpt, rpt)])

    @pl.when(sid == NS - 1)
    def _zero_tail():
        pltpu.sync_copy(zeros_hbm.at[pl.ds(0, N_PAD - N_ACC)],
                        out_hbm.at[pl.ds(cid * N_PAD + N_ACC, N_PAD - N_ACC)])


_edge_pass_cached = None


def _edge_pass(*args):
    global _edge_pass_cached
    if _edge_pass_cached is None:
        mesh = plsc.VectorSubcoreMesh(core_axis_name="c",
                                      subcore_axis_name="s")
        _edge_pass_cached = pl.kernel(
            _edge_pass_body,
            out_type=jax.ShapeDtypeStruct((NC * N_PAD, D), jnp.float32),
            mesh=mesh,
            scratch_types=(
                [pltpu.VMEM((2 * K,), jnp.int32)] * 3
                + [pltpu.VMEM((K,), jnp.int32)] * 3
                + [pltpu.VMEM((K, D), jnp.float32)] * 6
                + [pltpu.VMEM_SHARED((N_ACC, D), jnp.float32)]
                + [pltpu.SemaphoreType.DMA] * 9
            ),
        )
    return _edge_pass_cached(*args)


def _uv_body(ls_ref, wcat_ref, bmsg_ref, u_ref, v_ref):
    uv = jnp.dot(ls_ref[...], wcat_ref[...],
                 preferred_element_type=jnp.float32)
    u_ref[...] = uv[:, :D] + bmsg_ref[...]
    v_ref[...] = uv[:, D:]


def _uv_call(ls, wcat, bmsg):
    return pl.pallas_call(
        _uv_body,
        grid=(N_PAD // BN,),
        in_specs=[
            pl.BlockSpec((BN, D), lambda i: (i, 0)),
            pl.BlockSpec((D, 2 * D), lambda i: (0, 0)),
            pl.BlockSpec((1, D), lambda i: (0, 0)),
        ],
        out_specs=[
            pl.BlockSpec((BN, D), lambda i: (i, 0)),
            pl.BlockSpec((BN, D), lambda i: (i, 0)),
        ],
        out_shape=[
            jax.ShapeDtypeStruct((N_PAD, D), jnp.float32),
            jax.ShapeDtypeStruct((N_PAD, D), jnp.float32),
        ],
    )(ls, wcat, bmsg)


def _node_body(ls_ref, agga_ref, aggb_ref, w1t_ref, w1b_ref, b1_ref,
               w2_ref, b2_ref, wcat_ref, bmsg_ref,
               ls_out, u_out, v_out):
    agg = agga_ref[...] + aggb_ref[...]
    h = _selu(jnp.dot(ls_ref[...], w1t_ref[...],
                      preferred_element_type=jnp.float32)
              + jnp.dot(agg, w1b_ref[...],
                        preferred_element_type=jnp.float32)
              + b1_ref[...])
    ls_new = _selu(jnp.dot(h, w2_ref[...],
                           preferred_element_type=jnp.float32) + b2_ref[...])
    ls_out[...] = ls_new
    uv = jnp.dot(ls_new, wcat_ref[...], preferred_element_type=jnp.float32)
    u_out[...] = uv[:, :D] + bmsg_ref[...]
    v_out[...] = uv[:, D:]


def _node_call(ls, agg2, w1t, w1b, b1, w2, b2, wcat, bmsg):
    nb = N_PAD // BN
    return pl.pallas_call(
        _node_body,
        grid=(nb,),
        in_specs=[
            pl.BlockSpec((BN, D), lambda i: (i, 0)),
            pl.BlockSpec((BN, D), lambda i: (i, 0)),
            pl.BlockSpec((BN, D), lambda i, _nb=nb: (_nb + i, 0)),
            pl.BlockSpec((D, D), lambda i: (0, 0)),
            pl.BlockSpec((D, D), lambda i: (0, 0)),
            pl.BlockSpec((1, D), lambda i: (0, 0)),
            pl.BlockSpec((D, D), lambda i: (0, 0)),
            pl.BlockSpec((1, D), lambda i: (0, 0)),
            pl.BlockSpec((D, 2 * D), lambda i: (0, 0)),
            pl.BlockSpec((1, D), lambda i: (0, 0)),
        ],
        out_specs=[
            pl.BlockSpec((BN, D), lambda i: (i, 0)),
            pl.BlockSpec((BN, D), lambda i: (i, 0)),
            pl.BlockSpec((BN, D), lambda i: (i, 0)),
        ],
        out_shape=[
            jax.ShapeDtypeStruct((N_PAD, D), jnp.float32),
            jax.ShapeDtypeStruct((N_PAD, D), jnp.float32),
            jax.ShapeDtypeStruct((N_PAD, D), jnp.float32),
        ],
    )(ls, agg2, agg2, w1t, w1b, b1, w2, b2, wcat, bmsg)


def _readout_body(ls_ref, gid_ref, wr1_ref, br1_ref, wr2_ref, br2_ref,
                  wr3_ref, out_ref, acc_ref):
    i = pl.program_id(0)

    @pl.when(i == 0)
    def _init():
        acc_ref[...] = jnp.zeros_like(acc_ref)

    ids = gid_ref[0]
    onehot = (lax.broadcasted_iota(jnp.int32, (G, BN), 0) == ids
              ).astype(jnp.float32)
    acc_ref[...] += jnp.dot(onehot, ls_ref[...],
                            preferred_element_type=jnp.float32)

    @pl.when(i == pl.num_programs(0) - 1)
    def _fin():
        r = _selu(jnp.dot(acc_ref[...], wr1_ref[...],
                          preferred_element_type=jnp.float32) + br1_ref[...])
        r = _selu(jnp.dot(r, wr2_ref[...],
                          preferred_element_type=jnp.float32) + br2_ref[...])
        out_ref[...] = jnp.sum(r * wr3_ref[...], axis=1, keepdims=True) + \
            jnp.zeros((G, D), jnp.float32)


def _readout_call(ls, gid3, wr1, br1, wr2, br2, wr3row):
    return pl.pallas_call(
        _readout_body,
        grid=(N_PAD // BN,),
        in_specs=[
            pl.BlockSpec((BN, D), lambda i: (i, 0)),
            pl.BlockSpec((1, 1, BN), lambda i: (i, 0, 0)),
            pl.BlockSpec((D, R), lambda i: (0, 0)),
            pl.BlockSpec((1, R), lambda i: (0, 0)),
            pl.BlockSpec((R, R), lambda i: (0, 0)),
            pl.BlockSpec((1, R), lambda i: (0, 0)),
            pl.BlockSpec((1, R), lambda i: (0, 0)),
        ],
        out_specs=pl.BlockSpec((G, D), lambda i: (0, 0)),
        out_shape=jax.ShapeDtypeStruct((G, D), jnp.float32),
        scratch_shapes=[pltpu.VMEM((G, D), jnp.float32)],
    )(ls, gid3, wr1, br1, wr2, br2, wr3row)


def kernel(states_action, states_graph_ids, states_first, states_second,
           sates_num_edges, W_msg, b_msg, W_s1, b_s1, W_s2, b_s2,
           W_r1, b_r1, W_r2, b_r2, W_r3, b_r3):
    ls = jnp.pad(states_action, ((0, N_PAD - N), (0, 0)))
    first_p = jnp.concatenate(
        [states_first, jnp.zeros((E_PAD - E,), jnp.int32)])
    second_p = jnp.concatenate(
        [states_second, jnp.full((E_PAD - E,), DUMMY, jnp.int32)])
    eidx = jnp.concatenate(
        [first_p.reshape(NW, NB, 1, K), second_p.reshape(NW, NB, 1, K)],
        axis=2).reshape(-1)
    gid3 = jnp.pad(states_graph_ids, (0, N_PAD - N),
                   constant_values=G).reshape(N_PAD // BN, 1, BN)
    zeros = jnp.zeros((N_PAD, D), jnp.float32)

    wcat = jnp.concatenate([W_msg[:D], W_msg[D:]], axis=1)
    bmsg = b_msg.reshape(1, D)
    w1t, w1b = W_s1[:D], W_s1[D:]
    b1 = b_s1.reshape(1, D)
    b2 = b_s2.reshape(1, D)
    br1 = b_r1.reshape(1, R)
    br2 = b_r2.reshape(1, R)
    wr3row = W_r3.reshape(1, R)

    u, v = _uv_call(ls, wcat, bmsg)
    for _ in range(T):
        agg2 = _edge_pass(u, v, eidx, zeros)
        ls, u, v = _node_call(ls, agg2, w1t, w1b, b1, W_s2, b2, wcat, bmsg)

    out = _readout_call(ls, gid3, W_r1, br1, W_r2, br2, wr3row)
    r = out[:, :1] + b_r3
    return r + 0.0 * jnp.asarray(sates_num_edges, dtype=r.dtype)
